# baseline (speedup 1.0000x reference)
import jax
import jax.numpy as jnp
from jax.experimental import pallas as pl

_ROWS = 1024
_COLS = 27278
_BLOCK_ROWS = 64


def _mse_block_kernel(out_ref, tgt_ref, sum_ref, cnt_ref):
    i = pl.program_id(0)
    o = out_ref[...]
    t = tgt_ref[...]
    mask = t != -1.0
    d = o - t
    sq = jnp.where(mask, d * d, 0.0)
    s = jnp.sum(sq, keepdims=True)
    c = jnp.sum(mask.astype(jnp.float32), keepdims=True)

    @pl.when(i == 0)
    def _init():
        sum_ref[...] = s
        cnt_ref[...] = c

    @pl.when(i != 0)
    def _acc():
        sum_ref[...] += s
        cnt_ref[...] += c


def kernel(output, target):
    grid = (_ROWS // _BLOCK_ROWS,)
    in_spec = pl.BlockSpec((_BLOCK_ROWS, _COLS), lambda i: (i, 0))
    out_spec = pl.BlockSpec((1, 1), lambda i: (0, 0))
    loss_sum, n_obs = pl.pallas_call(
        _mse_block_kernel,
        grid=grid,
        in_specs=[in_spec, in_spec],
        out_specs=[out_spec, out_spec],
        out_shape=[
            jax.ShapeDtypeStruct((1, 1), jnp.float32),
            jax.ShapeDtypeStruct((1, 1), jnp.float32),
        ],
    )(output, target)
    return loss_sum[0, 0] / n_obs[0, 0]
